# baseline (speedup 1.0000x reference)
import jax
import jax.numpy as jnp
from jax import lax
from jax.experimental import pallas as pl
from jax.experimental.pallas import tpu as pltpu
from jax.experimental.pallas import tpu_sc as plsc

D_MODEL = 64
SCALE = 8.0
NC, NS = 2, 16
NW = NC * NS
LANES = 16

VOCAB = 1000000
TCBLK = 512


def _iota16():
    return lax.iota(jnp.int32, 16)


def _tc_transpose_body(tT_ref, o_ref):
    o_ref[:, :D_MODEL] = tT_ref[...].T * SCALE


def _gather_body(xT_hbm, scratch_hbm, out_hbm, xv, rows_v, obuf, sem_g, sem_w):
    wid = lax.axis_index("s") * NC + lax.axis_index("c")
    b0 = wid * 128
    n_cp = (xv.shape[0] * 8) // 2

    for cb in range(xv.shape[0]):
        pltpu.sync_copy(
            xT_hbm.at[pl.ds(cb * 8, 8), pl.ds(b0, 128)], xv.at[cb]
        )

    def issue_gather(cp):
        sl = cp & 1
        for h in range(2):
            c = cp * 2 + h
            pltpu.async_copy(
                scratch_hbm.at[xv.at[c >> 3, c & 7]],
                rows_v.at[pl.ds((sl * 2 + h) * 128, 128)],
                sem_g.at[sl],
            )

    def wait_gather(sl):
        for h in range(2):
            pltpu.make_async_copy(
                scratch_hbm.at[pl.ds(0, 128)],
                rows_v.at[pl.ds((sl * 2 + h) * 128, 128)],
                sem_g.at[sl],
            ).wait()

    def wait_write(sl):
        for h in range(2):
            pltpu.make_async_copy(
                obuf.at[pl.ds((sl * 2 + h) * 64, 64)],
                out_hbm.at[0, pl.ds(0, 64), pl.ds(b0, 128)],
                sem_w.at[sl],
            ).wait()

    issue_gather(0)

    @pl.loop(0, n_cp)
    def _colpair(cp):
        sl = cp & 1

        @pl.when(cp + 1 < n_cp)
        def _():
            issue_gather(cp + 1)

        wait_gather(sl)

        @pl.when(cp >= 2)
        def _():
            wait_write(sl)

        for h in range(2):
            rbase = (sl * 2 + h) * 128
            obase = (sl * 2 + h) * 64

            @plsc.parallel_loop(0, 128, unroll=16)
            def _row(b):
                bcol = jnp.full((16,), b, dtype=jnp.int32)
                for jb in range(D_MODEL // LANES):
                    vals = rows_v[rbase + b, pl.ds(jb * LANES, LANES)]
                    jrows = obase + jb * LANES + _iota16()
                    plsc.store_scatter(obuf, [jrows, bcol], vals)

        for h in range(2):
            c = cp * 2 + h
            pltpu.async_copy(
                obuf.at[pl.ds((sl * 2 + h) * 64, 64)],
                out_hbm.at[c, pl.ds(0, 64), pl.ds(b0, 128)],
                sem_w.at[sl],
            )

    wait_write(0)
    wait_write(1)


def kernel(x, table):
    n_b, n_c = x.shape
    mesh = plsc.VectorSubcoreMesh(core_axis_name="c", subcore_axis_name="s")
    params = pltpu.CompilerParams(
        use_tc_tiling_on_sc=True, needs_layout_passes=False
    )

    tT = table.T

    scratch = pl.pallas_call(
        _tc_transpose_body,
        grid=(pl.cdiv(VOCAB, TCBLK),),
        in_specs=[pl.BlockSpec((D_MODEL, TCBLK), lambda i: (0, i))],
        out_specs=pl.BlockSpec((TCBLK, 128), lambda i: (i, 0)),
        out_shape=jax.ShapeDtypeStruct((VOCAB, 128), jnp.float32),
    )(tT)

    outT = pl.kernel(
        _gather_body,
        out_type=jax.ShapeDtypeStruct((n_c, D_MODEL, n_b), jnp.float32),
        mesh=mesh,
        scratch_types=[
            pltpu.VMEM((n_c // 8, 8, 128), jnp.int32),
            pltpu.VMEM((4 * 128, 128), jnp.float32),
            pltpu.VMEM((4 * D_MODEL, 128), jnp.float32),
            pltpu.SemaphoreType.DMA((2,)),
            pltpu.SemaphoreType.DMA((2,)),
        ],
        compiler_params=params,
    )(x.T, scratch)

    return outT.transpose(2, 0, 1)


# speedup vs baseline: 1.5925x; 1.5925x over previous
import jax
import jax.numpy as jnp
from jax import lax
from jax.experimental import pallas as pl
from jax.experimental.pallas import tpu as pltpu
from jax.experimental.pallas import tpu_sc as plsc

D_MODEL = 64
SCALE = 8.0
NC, NS = 2, 16
NW = NC * NS
LANES = 16

VOCAB = 1000000
TCBLK = 2048


def _iota16():
    return lax.iota(jnp.int32, 16)


def _tc_transpose_body(tT_ref, o_ref):
    eye = jnp.eye(D_MODEL, dtype=jnp.float32) * SCALE
    o_ref[:, :D_MODEL] = jax.lax.dot_general(
        tT_ref[...], eye, (((0,), (0,)), ((), ())),
        preferred_element_type=jnp.float32,
    )


def _gather_body(xT_hbm, scratch_hbm, out_hbm, xv, rows_v, obuf, sem_g, sem_w):
    wid = lax.axis_index("s") * NC + lax.axis_index("c")
    b0 = wid * 128
    n_cp = (xv.shape[0] * 8) // 2

    for cb in range(xv.shape[0]):
        pltpu.sync_copy(
            xT_hbm.at[pl.ds(cb * 8, 8), pl.ds(b0, 128)], xv.at[cb]
        )

    def issue_gather(cp):
        sl = cp & 1
        for h in range(2):
            c = cp * 2 + h
            pltpu.async_copy(
                scratch_hbm.at[xv.at[c >> 3, c & 7]],
                rows_v.at[pl.ds((sl * 2 + h) * 128, 128)],
                sem_g.at[sl],
            )

    def wait_gather(sl):
        for h in range(2):
            pltpu.make_async_copy(
                scratch_hbm.at[pl.ds(0, 128)],
                rows_v.at[pl.ds((sl * 2 + h) * 128, 128)],
                sem_g.at[sl],
            ).wait()

    def wait_write(sl):
        for h in range(2):
            pltpu.make_async_copy(
                obuf.at[pl.ds((sl * 2 + h) * 64, 64)],
                out_hbm.at[0, pl.ds(0, 64), pl.ds(b0, 128)],
                sem_w.at[sl],
            ).wait()

    issue_gather(0)

    @pl.loop(0, n_cp)
    def _colpair(cp):
        sl = cp & 1

        @pl.when(cp + 1 < n_cp)
        def _():
            issue_gather(cp + 1)

        wait_gather(sl)

        @pl.when(cp >= 2)
        def _():
            wait_write(sl)

        for h in range(2):
            rbase = (sl * 2 + h) * 128
            obase = (sl * 2 + h) * 64

            @plsc.parallel_loop(0, 128, unroll=16)
            def _row(b):
                bcol = jnp.full((16,), b, dtype=jnp.int32)
                for jb in range(D_MODEL // LANES):
                    vals = rows_v[rbase + b, pl.ds(jb * LANES, LANES)]
                    jrows = obase + jb * LANES + _iota16()
                    plsc.store_scatter(obuf, [jrows, bcol], vals)

        for h in range(2):
            c = cp * 2 + h
            pltpu.async_copy(
                obuf.at[pl.ds((sl * 2 + h) * 64, 64)],
                out_hbm.at[c, pl.ds(0, 64), pl.ds(b0, 128)],
                sem_w.at[sl],
            )

    wait_write(0)
    wait_write(1)


def kernel(x, table):
    n_b, n_c = x.shape
    mesh = plsc.VectorSubcoreMesh(core_axis_name="c", subcore_axis_name="s")
    params = pltpu.CompilerParams(
        use_tc_tiling_on_sc=True, needs_layout_passes=False
    )

    tT = table.T

    scratch = pl.pallas_call(
        _tc_transpose_body,
        grid=(pl.cdiv(VOCAB, TCBLK),),
        in_specs=[pl.BlockSpec((D_MODEL, TCBLK), lambda i: (0, i))],
        out_specs=pl.BlockSpec((TCBLK, 128), lambda i: (i, 0)),
        out_shape=jax.ShapeDtypeStruct((VOCAB, 128), jnp.float32),
    )(tT)

    outT = pl.kernel(
        _gather_body,
        out_type=jax.ShapeDtypeStruct((n_c, D_MODEL, n_b), jnp.float32),
        mesh=mesh,
        scratch_types=[
            pltpu.VMEM((n_c // 8, 8, 128), jnp.int32),
            pltpu.VMEM((4 * 128, 128), jnp.float32),
            pltpu.VMEM((4 * D_MODEL, 128), jnp.float32),
            pltpu.SemaphoreType.DMA((2,)),
            pltpu.SemaphoreType.DMA((2,)),
        ],
        compiler_params=params,
    )(x.T, scratch)

    return outT.transpose(2, 0, 1)


# speedup vs baseline: 1.7989x; 1.1296x over previous
import jax
import jax.numpy as jnp
from jax import lax
from jax.experimental import pallas as pl
from jax.experimental.pallas import tpu as pltpu
from jax.experimental.pallas import tpu_sc as plsc

D_MODEL = 64
SCALE = 8.0
NC, NS = 2, 16
NW = NC * NS
LANES = 16

VOCAB = 1000000
TCBLK = 4096


def _iota16():
    return lax.iota(jnp.int32, 16)


def _tc_transpose_body(tT_ref, o_ref):
    eye = jnp.eye(D_MODEL, dtype=jnp.float32)
    o_ref[:, :D_MODEL] = jax.lax.dot_general(
        tT_ref[...], eye, (((0,), (0,)), ((), ())),
        preferred_element_type=jnp.float32,
    ) * SCALE


def _gather_body(xT_hbm, scratch_hbm, out_hbm, xv, rows_v, obuf, sem_g, sem_w):
    wid = lax.axis_index("s") * NC + lax.axis_index("c")
    b0 = wid * 128
    n_cp = (xv.shape[0] * 8) // 2

    for cb in range(xv.shape[0]):
        pltpu.sync_copy(
            xT_hbm.at[pl.ds(cb * 8, 8), pl.ds(b0, 128)], xv.at[cb]
        )

    def issue_gather(cp):
        sl = cp & 1
        for h in range(2):
            c = cp * 2 + h
            pltpu.async_copy(
                scratch_hbm.at[xv.at[c >> 3, c & 7]],
                rows_v.at[pl.ds((sl * 2 + h) * 128, 128)],
                sem_g.at[sl],
            )

    def wait_gather(sl):
        for h in range(2):
            pltpu.make_async_copy(
                scratch_hbm.at[pl.ds(0, 128)],
                rows_v.at[pl.ds((sl * 2 + h) * 128, 128)],
                sem_g.at[sl],
            ).wait()

    def wait_write(sl):
        for h in range(2):
            pltpu.make_async_copy(
                obuf.at[pl.ds((sl * 2 + h) * 64, 64)],
                out_hbm.at[0, pl.ds(0, 64), pl.ds(b0, 128)],
                sem_w.at[sl],
            ).wait()

    issue_gather(0)

    @pl.loop(0, n_cp)
    def _colpair(cp):
        sl = cp & 1

        @pl.when(cp + 1 < n_cp)
        def _():
            issue_gather(cp + 1)

        wait_gather(sl)

        @pl.when(cp >= 2)
        def _():
            wait_write(sl)

        for h in range(2):
            rbase = (sl * 2 + h) * 128
            obase = (sl * 2 + h) * 64

            @plsc.parallel_loop(0, 128, unroll=16)
            def _row(b):
                bcol = jnp.full((16,), b, dtype=jnp.int32)
                for jb in range(D_MODEL // LANES):
                    vals = rows_v[rbase + b, pl.ds(jb * LANES, LANES)]
                    jrows = obase + jb * LANES + _iota16()
                    plsc.store_scatter(obuf, [jrows, bcol], vals)

        for h in range(2):
            c = cp * 2 + h
            pltpu.async_copy(
                obuf.at[pl.ds((sl * 2 + h) * 64, 64)],
                out_hbm.at[c, pl.ds(0, 64), pl.ds(b0, 128)],
                sem_w.at[sl],
            )

    wait_write(0)
    wait_write(1)


def kernel(x, table):
    n_b, n_c = x.shape
    mesh = plsc.VectorSubcoreMesh(core_axis_name="c", subcore_axis_name="s")
    params = pltpu.CompilerParams(
        use_tc_tiling_on_sc=True, needs_layout_passes=False
    )

    tT = table.T

    scratch = pl.pallas_call(
        _tc_transpose_body,
        grid=(pl.cdiv(VOCAB, TCBLK),),
        in_specs=[pl.BlockSpec((D_MODEL, TCBLK), lambda i: (0, i))],
        out_specs=pl.BlockSpec((TCBLK, 128), lambda i: (i, 0)),
        out_shape=jax.ShapeDtypeStruct((VOCAB, 128), jnp.float32),
    )(tT)

    outT = pl.kernel(
        _gather_body,
        out_type=jax.ShapeDtypeStruct((n_c, D_MODEL, n_b), jnp.float32),
        mesh=mesh,
        scratch_types=[
            pltpu.VMEM((n_c // 8, 8, 128), jnp.int32),
            pltpu.VMEM((4 * 128, 128), jnp.float32),
            pltpu.VMEM((4 * D_MODEL, 128), jnp.float32),
            pltpu.SemaphoreType.DMA((2,)),
            pltpu.SemaphoreType.DMA((2,)),
        ],
        compiler_params=params,
    )(x.T, scratch)

    return outT.transpose(2, 0, 1)


# speedup vs baseline: 2.0754x; 1.1537x over previous
import jax
import jax.numpy as jnp
from jax import lax
from jax.experimental import pallas as pl
from jax.experimental.pallas import tpu as pltpu
from jax.experimental.pallas import tpu_sc as plsc

D_MODEL = 64
SCALE = 8.0
NC, NS = 2, 16
NW = NC * NS
LANES = 16

VOCAB = 1000000
TCBLK = 4096


def _iota16():
    return lax.iota(jnp.int32, 16)


def _tc_transpose_body(tT_ref, o_ref):
    eye = jnp.eye(D_MODEL, dtype=jnp.float32)
    o_ref[:, :D_MODEL] = jax.lax.dot_general(
        tT_ref[...], eye, (((0,), (0,)), ((), ())),
        preferred_element_type=jnp.float32,
    ) * SCALE


def _gather_body(xT_hbm, scratch_hbm, out2_hbm, xv, rows_v, sem_g, sem_w):
    wid = lax.axis_index("s") * NC + lax.axis_index("c")
    b0 = wid * 128
    n_c = xv.shape[0] * 8

    for cb in range(xv.shape[0]):
        pltpu.sync_copy(
            xT_hbm.at[pl.ds(cb * 8, 8), pl.ds(b0, 128)], xv.at[cb]
        )

    def issue_gather(c):
        sl = c & 3
        pltpu.async_copy(
            scratch_hbm.at[xv.at[c >> 3, c & 7]],
            rows_v.at[pl.ds(sl * 128, 128)],
            sem_g.at[sl],
        )

    def wait_gather(sl):
        pltpu.make_async_copy(
            scratch_hbm.at[pl.ds(0, 128)],
            rows_v.at[pl.ds(sl * 128, 128)],
            sem_g.at[sl],
        ).wait()

    def issue_write(c):
        sl = c & 3
        pltpu.async_copy(
            rows_v.at[pl.ds(sl * 128, 128)],
            out2_hbm.at[pl.ds(c * 4096 + b0, 128)],
            sem_w.at[sl],
        )

    def wait_write(sl):
        pltpu.make_async_copy(
            rows_v.at[pl.ds(sl * 128, 128)],
            out2_hbm.at[pl.ds(0, 128)],
            sem_w.at[sl],
        ).wait()

    issue_gather(0)
    issue_gather(1)

    @pl.loop(0, n_c)
    def _col(c):
        sl = c & 3

        @pl.when(c + 2 < n_c)
        def _():
            @pl.when(c >= 2)
            def _():
                wait_write((c + 2) & 3)

            issue_gather(c + 2)

        wait_gather(sl)
        issue_write(c)

    wait_write(0)
    wait_write(1)
    wait_write(2)
    wait_write(3)


def _tc_untranspose_body(in_ref, o_ref):
    eye = jnp.eye(D_MODEL, dtype=jnp.float32)
    o_ref[0] = jax.lax.dot_general(
        eye, in_ref[:, :D_MODEL], (((1,), (1,)), ((), ())),
        preferred_element_type=jnp.float32,
    )


def kernel(x, table):
    n_b, n_c = x.shape
    mesh = plsc.VectorSubcoreMesh(core_axis_name="c", subcore_axis_name="s")
    params = pltpu.CompilerParams(
        use_tc_tiling_on_sc=True, needs_layout_passes=False
    )

    tT = table.T

    scratch = pl.pallas_call(
        _tc_transpose_body,
        grid=(pl.cdiv(VOCAB, TCBLK),),
        in_specs=[pl.BlockSpec((D_MODEL, TCBLK), lambda i: (0, i))],
        out_specs=pl.BlockSpec((TCBLK, 128), lambda i: (i, 0)),
        out_shape=jax.ShapeDtypeStruct((VOCAB, 128), jnp.float32),
    )(tT)

    out2 = pl.kernel(
        _gather_body,
        out_type=jax.ShapeDtypeStruct((n_b * n_c, 128), jnp.float32),
        mesh=mesh,
        scratch_types=[
            pltpu.VMEM((n_c // 8, 8, 128), jnp.int32),
            pltpu.VMEM((4 * 128, 128), jnp.float32),
            pltpu.SemaphoreType.DMA((4,)),
            pltpu.SemaphoreType.DMA((4,)),
        ],
        compiler_params=params,
    )(x.T, scratch)

    outT = pl.pallas_call(
        _tc_untranspose_body,
        grid=(n_c,),
        in_specs=[pl.BlockSpec((n_b, 128), lambda i: (i, 0))],
        out_specs=pl.BlockSpec((1, D_MODEL, n_b), lambda i: (i, 0, 0)),
        out_shape=jax.ShapeDtypeStruct((n_c, D_MODEL, n_b), jnp.float32),
    )(out2)

    return outT.transpose(2, 0, 1)
